# Initial kernel scaffold; baseline (speedup 1.0000x reference)
import jax
import jax.numpy as jnp
from jax.experimental import pallas as pl


def kernel(x, y, W1, b1, Wg2, We2, be2, Wg3, We3, be3):
    raise NotImplementedError("write your pallas kernel here")


# speedup vs baseline: 1.2857x; 1.2857x over previous
import functools

import jax
import jax.numpy as jnp
from jax import lax
from jax.experimental import pallas as pl
from jax.experimental.pallas import tpu as pltpu
from jax.experimental.pallas import tpu_sc as plsc

B, S, D, E, CAP = 2, 2048, 768, 64, 80
T = B * S
SLOTS = E * CAP
TRASH = SLOTS
SMAP = SLOTS + 16
CHUNK = 512
NCHUNK = T // CHUNK

NW = 32
ROWS_PER_TILE = SLOTS // NW
GCH = 80
TOK_PER_TILE = T // NW


def _lin_body(x_ref, w_ref, b_ref, o_ref):
    o_ref[...] = (
        jnp.dot(x_ref[...], w_ref[...], preferred_element_type=jnp.float32)
        + b_ref[...]
    )


def _linear(x2d, W1, b1):
    return pl.pallas_call(
        _lin_body,
        grid=(NCHUNK,),
        in_specs=[
            pl.BlockSpec((CHUNK, D), lambda i: (i, 0)),
            pl.BlockSpec((D, D), lambda i: (0, 0)),
            pl.BlockSpec((1, D), lambda i: (0, 0)),
        ],
        out_specs=pl.BlockSpec((CHUNK, D), lambda i: (i, 0)),
        out_shape=jax.ShapeDtypeStruct((T, D), jnp.float32),
    )(x2d, W1, b1.reshape(1, D))


def _route_core(i, h, wg_ref, locs_ref, locg_ref, gate_ref, run_ref):
    logits = jnp.dot(h, wg_ref[...], preferred_element_type=jnp.float32)
    m = jnp.max(logits, axis=-1, keepdims=True)
    pmax = 1.0 / jnp.sum(jnp.exp(logits - m), axis=-1, keepdims=True)
    eidx = lax.broadcasted_iota(jnp.int32, (CHUNK, E), 1)
    idx = jnp.min(jnp.where(logits == m, eidx, E), axis=-1, keepdims=True)
    maskf = (eidx == idx).astype(jnp.float32)
    r = lax.broadcasted_iota(jnp.int32, (CHUNK, CHUNK), 0)
    c = lax.broadcasted_iota(jnp.int32, (CHUNK, CHUNK), 1)
    tril = (r >= c).astype(jnp.float32)

    @pl.when(i == 0)
    def _():
        run_ref[...] = jnp.zeros_like(run_ref)

    cum = (
        jnp.dot(tril, maskf, preferred_element_type=jnp.float32)
        + run_ref[...]
    )
    pos = jnp.sum(maskf * cum, axis=-1, keepdims=True) - 1.0
    run_ref[...] = run_ref[...] + jnp.sum(maskf, axis=0, keepdims=True)
    keep = pos < float(CAP)
    loc = idx * CAP + pos.astype(jnp.int32)
    tid = lax.broadcasted_iota(jnp.int32, (CHUNK, 1), 0)
    locs_ref[...] = jnp.where(keep, loc, TRASH + (tid & 15))
    locg_ref[...] = jnp.where(keep, loc, 0)
    gate_ref[...] = jnp.where(keep, pmax, 0.0)


def _route2_body(hin_ref, wg_ref, locs_ref, locg_ref, gate_ref, run_ref):
    _route_core(pl.program_id(0), hin_ref[...], wg_ref,
                locs_ref, locg_ref, gate_ref, run_ref)


def _route3_body(comb_ref, gprev_ref, wg_ref,
                 locs_ref, locg_ref, gate_ref, h2_ref, run_ref):
    h = comb_ref[...] * gprev_ref[...]
    h2_ref[...] = h
    _route_core(pl.program_id(0), h, wg_ref,
                locs_ref, locg_ref, gate_ref, run_ref)


_TOKVEC_SPEC = pl.BlockSpec((CHUNK, 1), lambda i: (i, 0))
_TOKVEC_I32 = jax.ShapeDtypeStruct((T, 1), jnp.int32)
_TOKVEC_F32 = jax.ShapeDtypeStruct((T, 1), jnp.float32)


def _route2(h, Wg):
    return pl.pallas_call(
        _route2_body,
        grid=(NCHUNK,),
        in_specs=[
            pl.BlockSpec((CHUNK, D), lambda i: (i, 0)),
            pl.BlockSpec((D, E), lambda i: (0, 0)),
        ],
        out_specs=[_TOKVEC_SPEC, _TOKVEC_SPEC, _TOKVEC_SPEC],
        out_shape=[_TOKVEC_I32, _TOKVEC_I32, _TOKVEC_F32],
        scratch_shapes=[pltpu.VMEM((1, E), jnp.float32)],
    )(h, Wg)


def _route3(comb, gate_prev, Wg):
    return pl.pallas_call(
        _route3_body,
        grid=(NCHUNK,),
        in_specs=[
            pl.BlockSpec((CHUNK, D), lambda i: (i, 0)),
            pl.BlockSpec((CHUNK, 1), lambda i: (i, 0)),
            pl.BlockSpec((D, E), lambda i: (0, 0)),
        ],
        out_specs=[_TOKVEC_SPEC, _TOKVEC_SPEC, _TOKVEC_SPEC,
                   pl.BlockSpec((CHUNK, D), lambda i: (i, 0))],
        out_shape=[_TOKVEC_I32, _TOKVEC_I32, _TOKVEC_F32,
                   jax.ShapeDtypeStruct((T, D), jnp.float32)],
        scratch_shapes=[pltpu.VMEM((1, E), jnp.float32)],
    )(comb, gate_prev, Wg)


def _expert_body(buf_ref, w_ref, b_ref, o_ref):
    o_ref[0] = (
        jnp.dot(buf_ref[0], w_ref[0], preferred_element_type=jnp.float32)
        + b_ref[0]
    )


def _expert(buf, We, be):
    return pl.pallas_call(
        _expert_body,
        grid=(E,),
        in_specs=[
            pl.BlockSpec((1, CAP, D), lambda e: (e, 0, 0)),
            pl.BlockSpec((1, D, D), lambda e: (e, 0, 0)),
            pl.BlockSpec((1, 1, D), lambda e: (e, 0, 0)),
        ],
        out_specs=pl.BlockSpec((1, CAP, D), lambda e: (e, 0, 0)),
        out_shape=jax.ShapeDtypeStruct((E, CAP, D), jnp.float32),
    )(buf, We, be.reshape(E, 1, D))


def _dispatch_body(h_hbm, locs_hbm, buf_hbm, loc_v, smap_v, idx_v, rows_v,
                   sem):
    wid = lax.axis_index("s") * 2 + lax.axis_index("c")
    pltpu.sync_copy(locs_hbm, loc_v)

    def initb(j, _):
        smap_v[pl.ds(j * 16, 16)] = jnp.zeros((16,), jnp.int32)
        return 0

    lax.fori_loop(0, SMAP // 16, initb, 0)

    def scat(j, _):
        idx16 = loc_v[pl.ds(j * 16, 16)]
        vals = lax.iota(jnp.int32, 16) + j * 16
        plsc.store_scatter(smap_v, [idx16], vals)
        return 0

    lax.fori_loop(0, T // 16, scat, 0)

    base = wid * ROWS_PER_TILE

    def gchunk(half, _):
        s0 = base + half * GCH

        def cp(j, _):
            idx_v[pl.ds(j * 16, 16)] = smap_v[pl.ds(s0 + j * 16, 16)]
            return 0

        lax.fori_loop(0, GCH // 16, cp, 0)
        pltpu.async_copy(h_hbm.at[idx_v], rows_v, sem).wait()
        pltpu.sync_copy(rows_v, buf_hbm.at[pl.ds(s0, GCH)])
        return 0

    lax.fori_loop(0, ROWS_PER_TILE // GCH, gchunk, 0)


def _combine_body(eo_hbm, locg_hbm, out_hbm, idx_v, rows_v, sem):
    wid = lax.axis_index("s") * 2 + lax.axis_index("c")
    base = wid * TOK_PER_TILE
    pltpu.sync_copy(locg_hbm.at[pl.ds(base, TOK_PER_TILE)], idx_v)
    pltpu.async_copy(eo_hbm.at[idx_v], rows_v, sem).wait()
    pltpu.sync_copy(rows_v, out_hbm.at[pl.ds(base, TOK_PER_TILE)])


@functools.cache
def _sc_kernels():
    mesh = plsc.VectorSubcoreMesh(core_axis_name="c", subcore_axis_name="s")
    params = pltpu.CompilerParams(needs_layout_passes=False)
    dispatch = pl.kernel(
        _dispatch_body,
        out_type=jax.ShapeDtypeStruct((SLOTS, D), jnp.float32),
        mesh=mesh,
        compiler_params=params,
        scratch_types=[
            pltpu.VMEM((T,), jnp.int32),
            pltpu.VMEM((SMAP,), jnp.int32),
            pltpu.VMEM((GCH,), jnp.int32),
            pltpu.VMEM((GCH, D), jnp.float32),
            pltpu.SemaphoreType.DMA,
        ],
    )
    combine = pl.kernel(
        _combine_body,
        out_type=jax.ShapeDtypeStruct((T, D), jnp.float32),
        mesh=mesh,
        compiler_params=params,
        scratch_types=[
            pltpu.VMEM((TOK_PER_TILE,), jnp.int32),
            pltpu.VMEM((TOK_PER_TILE, D), jnp.float32),
            pltpu.SemaphoreType.DMA,
        ],
    )
    return dispatch, combine


def _dispatch(h, locs):
    return _sc_kernels()[0](h, locs)


def _combine(eo, locg):
    return _sc_kernels()[1](eo, locg)


def _final_body(h_ref, comb_ref, gate_ref, y_ref, o_ref, acc_ref):
    i = pl.program_id(0)
    hid = h_ref[...] + comb_ref[...] * gate_ref[...]
    cs = jnp.sum(hid, axis=0, keepdims=True)
    rows8 = lax.broadcasted_iota(jnp.int32, (NCHUNK, 1), 0)
    upd = jnp.where(rows8 == i, 1.0, 0.0) * cs

    @pl.when(i == 0)
    def _():
        acc_ref[...] = upd

    @pl.when(i > 0)
    def _():
        acc_ref[...] = acc_ref[...] + upd

    @pl.when(i == NCHUNK - 1)
    def _():
        sums = acc_ref[...]
        b0 = jnp.sum(sums[0:4, :], axis=0, keepdims=True)
        b1 = jnp.sum(sums[4:8, :], axis=0, keepdims=True)
        sent = jnp.concatenate([b0, b1], axis=0) * (1.0 / S)
        m = jnp.max(sent, axis=-1, keepdims=True)
        lse = jnp.log(jnp.sum(jnp.exp(sent - m), axis=-1, keepdims=True)) + m
        logp = sent - lse
        col = lax.broadcasted_iota(jnp.int32, (B, D), 1)
        row = lax.broadcasted_iota(jnp.int32, (B, D), 0)
        ysel = jnp.where(row == 0, y_ref[0], y_ref[1])
        loss = -jnp.sum(jnp.where(col == ysel, logp, 0.0)) * (1.0 / B)
        o_ref[...] = jnp.full((8, 128), loss, jnp.float32)


def _final(h, comb3, gate3, y32):
    return pl.pallas_call(
        _final_body,
        grid=(NCHUNK,),
        in_specs=[
            pl.BlockSpec((CHUNK, D), lambda i: (i, 0)),
            pl.BlockSpec((CHUNK, D), lambda i: (i, 0)),
            pl.BlockSpec((CHUNK, 1), lambda i: (i, 0)),
            pl.BlockSpec(memory_space=pltpu.SMEM),
        ],
        out_specs=pl.BlockSpec((8, 128), lambda i: (0, 0)),
        out_shape=jax.ShapeDtypeStruct((8, 128), jnp.float32),
        scratch_shapes=[pltpu.VMEM((NCHUNK, D), jnp.float32)],
    )(h, comb3, gate3, y32)


def kernel(x, y, W1, b1, Wg2, We2, be2, Wg3, We3, be3):
    x2d = x.reshape(T, D)
    y32 = y.astype(jnp.int32)

    h = _linear(x2d, W1, b1)

    locs2, locg2, gate2 = _route2(h, Wg2)
    buf2 = _dispatch(h, locs2.reshape(T))
    eo2 = _expert(buf2.reshape(E, CAP, D), We2, be2)
    comb2 = _combine(eo2.reshape(SLOTS, D), locg2.reshape(T))

    locs3, locg3, gate3, h2 = _route3(comb2, gate2, Wg3)
    buf3 = _dispatch(h2, locs3.reshape(T))
    eo3 = _expert(buf3.reshape(E, CAP, D), We3, be3)
    comb3 = _combine(eo3.reshape(SLOTS, D), locg3.reshape(T))

    loss = _final(h, comb3, gate3, y32)
    return loss[0, 0]


# speedup vs baseline: 1.3233x; 1.0293x over previous
import functools

import jax
import jax.numpy as jnp
from jax import lax
from jax.experimental import pallas as pl
from jax.experimental.pallas import tpu as pltpu
from jax.experimental.pallas import tpu_sc as plsc

B, S, D, E, CAP = 2, 2048, 768, 64, 80
T = B * S
SLOTS = E * CAP
TRASH = SLOTS
SMAP_PAD = 9472
STRIPE = SMAP_PAD // 16
CHUNK = 512
NCHUNK = T // CHUNK

NW = 32
ROWS_PER_TILE = SLOTS // NW
HGCH = 80
TPS = T // 16
TOK_PER_TILE = T // NW


def _lin_body(x_ref, w_ref, b_ref, o_ref):
    o_ref[...] = (
        jnp.dot(x_ref[...], w_ref[...], preferred_element_type=jnp.float32)
        + b_ref[...]
    )


def _linear(x2d, W1, b1):
    return pl.pallas_call(
        _lin_body,
        grid=(NCHUNK,),
        in_specs=[
            pl.BlockSpec((CHUNK, D), lambda i: (i, 0)),
            pl.BlockSpec((D, D), lambda i: (0, 0)),
            pl.BlockSpec((1, D), lambda i: (0, 0)),
        ],
        out_specs=pl.BlockSpec((CHUNK, D), lambda i: (i, 0)),
        out_shape=jax.ShapeDtypeStruct((T, D), jnp.float32),
    )(x2d, W1, b1.reshape(1, D))


def _route_core(i, h, wg_ref, locs_ref, locg_ref, gate_ref, run_ref):
    logits = jnp.dot(h, wg_ref[...], preferred_element_type=jnp.float32)
    m = jnp.max(logits, axis=-1, keepdims=True)
    pmax = 1.0 / jnp.sum(jnp.exp(logits - m), axis=-1, keepdims=True)
    eidx = lax.broadcasted_iota(jnp.int32, (CHUNK, E), 1)
    idx = jnp.min(jnp.where(logits == m, eidx, E), axis=-1, keepdims=True)
    maskf = (eidx == idx).astype(jnp.float32)
    r = lax.broadcasted_iota(jnp.int32, (CHUNK, CHUNK), 0)
    c = lax.broadcasted_iota(jnp.int32, (CHUNK, CHUNK), 1)
    tril = (r >= c).astype(jnp.float32)

    @pl.when(i == 0)
    def _():
        run_ref[...] = jnp.zeros_like(run_ref)

    cum = (
        jnp.dot(tril, maskf, preferred_element_type=jnp.float32)
        + run_ref[...]
    )
    pos = jnp.sum(maskf * cum, axis=-1, keepdims=True) - 1.0
    run_ref[...] = run_ref[...] + jnp.sum(maskf, axis=0, keepdims=True)
    keep = pos < float(CAP)
    loc = idx * CAP + pos.astype(jnp.int32)
    tid = lax.broadcasted_iota(jnp.int32, (CHUNK, 1), 0)
    locs_ref[...] = jnp.where(keep, loc, TRASH + i * CHUNK + tid)
    locg_ref[...] = jnp.where(keep, loc, 0)
    gate_ref[...] = jnp.where(keep, pmax, 0.0)


def _route2_body(hin_ref, wg_ref, locs_ref, locg_ref, gate_ref, run_ref):
    _route_core(pl.program_id(0), hin_ref[...], wg_ref,
                locs_ref, locg_ref, gate_ref, run_ref)


def _route3_body(comb_ref, gprev_ref, wg_ref,
                 locs_ref, locg_ref, gate_ref, h2_ref, run_ref):
    h = comb_ref[...] * gprev_ref[...]
    h2_ref[...] = h
    _route_core(pl.program_id(0), h, wg_ref,
                locs_ref, locg_ref, gate_ref, run_ref)


_TOKVEC_SPEC = pl.BlockSpec((CHUNK, 1), lambda i: (i, 0))
_TOKVEC_I32 = jax.ShapeDtypeStruct((T, 1), jnp.int32)
_TOKVEC_F32 = jax.ShapeDtypeStruct((T, 1), jnp.float32)


def _route2(h, Wg):
    return pl.pallas_call(
        _route2_body,
        grid=(NCHUNK,),
        in_specs=[
            pl.BlockSpec((CHUNK, D), lambda i: (i, 0)),
            pl.BlockSpec((D, E), lambda i: (0, 0)),
        ],
        out_specs=[_TOKVEC_SPEC, _TOKVEC_SPEC, _TOKVEC_SPEC],
        out_shape=[_TOKVEC_I32, _TOKVEC_I32, _TOKVEC_F32],
        scratch_shapes=[pltpu.VMEM((1, E), jnp.float32)],
    )(h, Wg)


def _route3(comb, gate_prev, Wg):
    return pl.pallas_call(
        _route3_body,
        grid=(NCHUNK,),
        in_specs=[
            pl.BlockSpec((CHUNK, D), lambda i: (i, 0)),
            pl.BlockSpec((CHUNK, 1), lambda i: (i, 0)),
            pl.BlockSpec((D, E), lambda i: (0, 0)),
        ],
        out_specs=[_TOKVEC_SPEC, _TOKVEC_SPEC, _TOKVEC_SPEC,
                   pl.BlockSpec((CHUNK, D), lambda i: (i, 0))],
        out_shape=[_TOKVEC_I32, _TOKVEC_I32, _TOKVEC_F32,
                   jax.ShapeDtypeStruct((T, D), jnp.float32)],
        scratch_shapes=[pltpu.VMEM((1, E), jnp.float32)],
    )(comb, gate_prev, Wg)


def _expert_body(buf_ref, w_ref, b_ref, o_ref):
    o_ref[0] = (
        jnp.dot(buf_ref[0], w_ref[0], preferred_element_type=jnp.float32)
        + b_ref[0]
    )


def _expert(buf, We, be):
    return pl.pallas_call(
        _expert_body,
        grid=(E,),
        in_specs=[
            pl.BlockSpec((1, CAP, D), lambda e: (e, 0, 0)),
            pl.BlockSpec((1, D, D), lambda e: (e, 0, 0)),
            pl.BlockSpec((1, 1, D), lambda e: (e, 0, 0)),
        ],
        out_specs=pl.BlockSpec((1, CAP, D), lambda e: (e, 0, 0)),
        out_shape=jax.ShapeDtypeStruct((E, CAP, D), jnp.float32),
    )(buf, We, be.reshape(E, 1, D))


def _dispatch_body(h_hbm, locs_hbm, buf_hbm, zero_v, loc_v, tok_v,
                   idx_a, idx_b, rows_a, rows_b, smap_sh, sem_a, sem_b):
    cid = lax.axis_index("c")
    sid = lax.axis_index("s")
    wid = sid * 2 + cid

    def z(j, _):
        zero_v[pl.ds(j * 16, 16)] = jnp.zeros((16,), jnp.int32)
        return 0

    lax.fori_loop(0, STRIPE // 16, z, 0)
    pltpu.sync_copy(zero_v, smap_sh.at[pl.ds(sid * STRIPE, STRIPE)])
    pltpu.sync_copy(locs_hbm.at[pl.ds(sid * TPS, TPS)], loc_v)

    def tk(j, _):
        tok_v[pl.ds(j * 16, 16)] = lax.iota(jnp.int32, 16) + (sid * TPS + j * 16)
        return 0

    lax.fori_loop(0, TPS // 16, tk, 0)
    plsc.subcore_barrier()
    pltpu.sync_copy(tok_v, smap_sh.at[loc_v], add=True)
    plsc.subcore_barrier()

    base = wid * ROWS_PER_TILE
    pltpu.sync_copy(smap_sh.at[pl.ds(base, HGCH)], idx_a)
    pltpu.sync_copy(smap_sh.at[pl.ds(base + HGCH, HGCH)], idx_b)
    cpa = pltpu.async_copy(h_hbm.at[idx_a], rows_a, sem_a)
    cpb = pltpu.async_copy(h_hbm.at[idx_b], rows_b, sem_b)
    cpa.wait()
    wa = pltpu.async_copy(rows_a, buf_hbm.at[pl.ds(base, HGCH)], sem_a)
    cpb.wait()
    wb = pltpu.async_copy(rows_b, buf_hbm.at[pl.ds(base + HGCH, HGCH)], sem_b)
    wa.wait()
    wb.wait()


def _combine_body(eo_hbm, locg_hbm, out_hbm, idx_v, rows_v, sem):
    wid = lax.axis_index("s") * 2 + lax.axis_index("c")
    base = wid * TOK_PER_TILE
    pltpu.sync_copy(locg_hbm.at[pl.ds(base, TOK_PER_TILE)], idx_v)
    pltpu.async_copy(eo_hbm.at[idx_v], rows_v, sem).wait()
    pltpu.sync_copy(rows_v, out_hbm.at[pl.ds(base, TOK_PER_TILE)])


@functools.cache
def _sc_kernels():
    mesh = plsc.VectorSubcoreMesh(core_axis_name="c", subcore_axis_name="s")
    params = pltpu.CompilerParams(needs_layout_passes=False)
    dispatch = pl.kernel(
        _dispatch_body,
        out_type=jax.ShapeDtypeStruct((SLOTS, D), jnp.float32),
        mesh=mesh,
        compiler_params=params,
        scratch_types=[
            pltpu.VMEM((STRIPE,), jnp.int32),
            pltpu.VMEM((TPS,), jnp.int32),
            pltpu.VMEM((TPS,), jnp.int32),
            pltpu.VMEM((HGCH,), jnp.int32),
            pltpu.VMEM((HGCH,), jnp.int32),
            pltpu.VMEM((HGCH, D), jnp.float32),
            pltpu.VMEM((HGCH, D), jnp.float32),
            pltpu.VMEM_SHARED((SMAP_PAD,), jnp.int32),
            pltpu.SemaphoreType.DMA,
            pltpu.SemaphoreType.DMA,
        ],
    )
    combine = pl.kernel(
        _combine_body,
        out_type=jax.ShapeDtypeStruct((T, D), jnp.float32),
        mesh=mesh,
        compiler_params=params,
        scratch_types=[
            pltpu.VMEM((TOK_PER_TILE,), jnp.int32),
            pltpu.VMEM((TOK_PER_TILE, D), jnp.float32),
            pltpu.SemaphoreType.DMA,
        ],
    )
    return dispatch, combine


def _dispatch(h, locs):
    return _sc_kernels()[0](h, locs)


def _combine(eo, locg):
    return _sc_kernels()[1](eo, locg)


def _final_body(h_ref, comb_ref, gate_ref, y_ref, o_ref, acc_ref):
    i = pl.program_id(0)
    hid = h_ref[...] + comb_ref[...] * gate_ref[...]
    cs = jnp.sum(hid, axis=0, keepdims=True)
    rows8 = lax.broadcasted_iota(jnp.int32, (NCHUNK, 1), 0)
    upd = jnp.where(rows8 == i, 1.0, 0.0) * cs

    @pl.when(i == 0)
    def _():
        acc_ref[...] = upd

    @pl.when(i > 0)
    def _():
        acc_ref[...] = acc_ref[...] + upd

    @pl.when(i == NCHUNK - 1)
    def _():
        sums = acc_ref[...]
        b0 = jnp.sum(sums[0:4, :], axis=0, keepdims=True)
        b1 = jnp.sum(sums[4:8, :], axis=0, keepdims=True)
        sent = jnp.concatenate([b0, b1], axis=0) * (1.0 / S)
        m = jnp.max(sent, axis=-1, keepdims=True)
        lse = jnp.log(jnp.sum(jnp.exp(sent - m), axis=-1, keepdims=True)) + m
        logp = sent - lse
        col = lax.broadcasted_iota(jnp.int32, (B, D), 1)
        row = lax.broadcasted_iota(jnp.int32, (B, D), 0)
        ysel = jnp.where(row == 0, y_ref[0], y_ref[1])
        loss = -jnp.sum(jnp.where(col == ysel, logp, 0.0)) * (1.0 / B)
        o_ref[...] = jnp.full((8, 128), loss, jnp.float32)


def _final(h, comb3, gate3, y32):
    return pl.pallas_call(
        _final_body,
        grid=(NCHUNK,),
        in_specs=[
            pl.BlockSpec((CHUNK, D), lambda i: (i, 0)),
            pl.BlockSpec((CHUNK, D), lambda i: (i, 0)),
            pl.BlockSpec((CHUNK, 1), lambda i: (i, 0)),
            pl.BlockSpec(memory_space=pltpu.SMEM),
        ],
        out_specs=pl.BlockSpec((8, 128), lambda i: (0, 0)),
        out_shape=jax.ShapeDtypeStruct((8, 128), jnp.float32),
        scratch_shapes=[pltpu.VMEM((NCHUNK, D), jnp.float32)],
    )(h, comb3, gate3, y32)


def kernel(x, y, W1, b1, Wg2, We2, be2, Wg3, We3, be3):
    x2d = x.reshape(T, D)
    y32 = y.astype(jnp.int32)

    h = _linear(x2d, W1, b1)

    locs2, locg2, gate2 = _route2(h, Wg2)
    buf2 = _dispatch(h, locs2.reshape(T))
    eo2 = _expert(buf2.reshape(E, CAP, D), We2, be2)
    comb2 = _combine(eo2.reshape(SLOTS, D), locg2.reshape(T))

    locs3, locg3, gate3, h2 = _route3(comb2, gate2, Wg3)
    buf3 = _dispatch(h2, locs3.reshape(T))
    eo3 = _expert(buf3.reshape(E, CAP, D), We3, be3)
    comb3 = _combine(eo3.reshape(SLOTS, D), locg3.reshape(T))

    loss = _final(h, comb3, gate3, y32)
    return loss[0, 0]


# speedup vs baseline: 1.8851x; 1.4245x over previous
import functools

import jax
import jax.numpy as jnp
from jax import lax
from jax.experimental import pallas as pl
from jax.experimental.pallas import tpu as pltpu
from jax.experimental.pallas import tpu_sc as plsc

B, S, D, E, CAP = 2, 2048, 768, 64, 80
T = B * S
SLOTS = E * CAP
TRASH = SLOTS
SLOTS_PAD = SLOTS + CAP
CHUNK = 512
NCHUNK = T // CHUNK

NW = 32
TOK_PER_TILE = T // NW


def _lin_body(x_ref, w_ref, b_ref, o_ref):
    o_ref[...] = (
        jnp.dot(x_ref[...], w_ref[...], preferred_element_type=jnp.float32)
        + b_ref[...]
    )


def _linear(x2d, W1, b1):
    return pl.pallas_call(
        _lin_body,
        grid=(NCHUNK,),
        in_specs=[
            pl.BlockSpec((CHUNK, D), lambda i: (i, 0)),
            pl.BlockSpec((D, D), lambda i: (0, 0)),
            pl.BlockSpec((1, D), lambda i: (0, 0)),
        ],
        out_specs=pl.BlockSpec((CHUNK, D), lambda i: (i, 0)),
        out_shape=jax.ShapeDtypeStruct((T, D), jnp.float32),
    )(x2d, W1, b1.reshape(1, D))


def _route_core(i, h, wg_ref, locs_ref, locg_ref, gate_ref, run_ref):
    logits = jnp.dot(h, wg_ref[...], preferred_element_type=jnp.float32)
    m = jnp.max(logits, axis=-1, keepdims=True)
    pmax = 1.0 / jnp.sum(jnp.exp(logits - m), axis=-1, keepdims=True)
    eidx = lax.broadcasted_iota(jnp.int32, (CHUNK, E), 1)
    idx = jnp.min(jnp.where(logits == m, eidx, E), axis=-1, keepdims=True)
    maskf = (eidx == idx).astype(jnp.float32)
    r = lax.broadcasted_iota(jnp.int32, (CHUNK, CHUNK), 0)
    c = lax.broadcasted_iota(jnp.int32, (CHUNK, CHUNK), 1)
    tril = (r >= c).astype(jnp.float32)

    @pl.when(i == 0)
    def _():
        run_ref[...] = jnp.zeros_like(run_ref)

    cum = (
        jnp.dot(tril, maskf, preferred_element_type=jnp.float32)
        + run_ref[...]
    )
    pos = jnp.sum(maskf * cum, axis=-1, keepdims=True) - 1.0
    run_ref[...] = run_ref[...] + jnp.sum(maskf, axis=0, keepdims=True)
    keep = pos < float(CAP)
    loc = idx * CAP + pos.astype(jnp.int32)
    tid = lax.broadcasted_iota(jnp.int32, (CHUNK, 1), 0)
    locs_ref[...] = jnp.where(keep, loc, TRASH + (tid & 15))
    locg_ref[...] = jnp.where(keep, loc, 0)
    gate_ref[...] = jnp.where(keep, pmax, 0.0)


def _route2_body(hin_ref, wg_ref, locs_ref, locg_ref, gate_ref, run_ref):
    _route_core(pl.program_id(0), hin_ref[...], wg_ref,
                locs_ref, locg_ref, gate_ref, run_ref)


def _route3_body(comb_ref, gprev_ref, wg_ref,
                 locs_ref, locg_ref, gate_ref, h2_ref, run_ref):
    g = gprev_ref[...]
    h = jnp.where(g > 0.0, comb_ref[...] * g, 0.0)
    h2_ref[...] = h
    _route_core(pl.program_id(0), h, wg_ref,
                locs_ref, locg_ref, gate_ref, run_ref)


_TOKVEC_SPEC = pl.BlockSpec((CHUNK, 1), lambda i: (i, 0))
_TOKVEC_I32 = jax.ShapeDtypeStruct((T, 1), jnp.int32)
_TOKVEC_F32 = jax.ShapeDtypeStruct((T, 1), jnp.float32)


def _route2(h, Wg):
    return pl.pallas_call(
        _route2_body,
        grid=(NCHUNK,),
        in_specs=[
            pl.BlockSpec((CHUNK, D), lambda i: (i, 0)),
            pl.BlockSpec((D, E), lambda i: (0, 0)),
        ],
        out_specs=[_TOKVEC_SPEC, _TOKVEC_SPEC, _TOKVEC_SPEC],
        out_shape=[_TOKVEC_I32, _TOKVEC_I32, _TOKVEC_F32],
        scratch_shapes=[pltpu.VMEM((1, E), jnp.float32)],
    )(h, Wg)


def _route3(comb, gate_prev, Wg):
    return pl.pallas_call(
        _route3_body,
        grid=(NCHUNK,),
        in_specs=[
            pl.BlockSpec((CHUNK, D), lambda i: (i, 0)),
            pl.BlockSpec((CHUNK, 1), lambda i: (i, 0)),
            pl.BlockSpec((D, E), lambda i: (0, 0)),
        ],
        out_specs=[_TOKVEC_SPEC, _TOKVEC_SPEC, _TOKVEC_SPEC,
                   pl.BlockSpec((CHUNK, D), lambda i: (i, 0))],
        out_shape=[_TOKVEC_I32, _TOKVEC_I32, _TOKVEC_F32,
                   jax.ShapeDtypeStruct((T, D), jnp.float32)],
        scratch_shapes=[pltpu.VMEM((1, E), jnp.float32)],
    )(comb, gate_prev, Wg)


def _expert_body(buf_ref, w_ref, b_ref, o_ref):
    o_ref[0] = (
        jnp.dot(buf_ref[0], w_ref[0], preferred_element_type=jnp.float32)
        + b_ref[0]
    )


def _expert(buf, We, be):
    buf = buf.reshape(SLOTS_PAD // CAP, CAP, D)
    return pl.pallas_call(
        _expert_body,
        grid=(E,),
        in_specs=[
            pl.BlockSpec((1, CAP, D), lambda e: (e, 0, 0)),
            pl.BlockSpec((1, D, D), lambda e: (e, 0, 0)),
            pl.BlockSpec((1, 1, D), lambda e: (e, 0, 0)),
        ],
        out_specs=pl.BlockSpec((1, CAP, D), lambda e: (e, 0, 0)),
        out_shape=jax.ShapeDtypeStruct((E, CAP, D), jnp.float32),
    )(buf, We, be.reshape(E, 1, D))


def _dispatch_body(h_hbm, locs_hbm, buf_hbm, loc_v, rows_v, sem, sem2):
    cid = lax.axis_index("c")
    sid = lax.axis_index("s")
    wid = sid * 2 + cid
    base = wid * TOK_PER_TILE
    pltpu.sync_copy(locs_hbm.at[pl.ds(base, TOK_PER_TILE)], loc_v)
    pltpu.async_copy(h_hbm.at[pl.ds(base, TOK_PER_TILE)], rows_v, sem).wait()
    pltpu.async_copy(rows_v, buf_hbm.at[loc_v], sem2).wait()


def _combine_body(eo_hbm, locg_hbm, out_hbm, idx_v, rows_v, sem):
    wid = lax.axis_index("s") * 2 + lax.axis_index("c")
    base = wid * TOK_PER_TILE
    pltpu.sync_copy(locg_hbm.at[pl.ds(base, TOK_PER_TILE)], idx_v)
    pltpu.async_copy(eo_hbm.at[idx_v], rows_v, sem).wait()
    pltpu.sync_copy(rows_v, out_hbm.at[pl.ds(base, TOK_PER_TILE)])


@functools.cache
def _sc_kernels():
    mesh = plsc.VectorSubcoreMesh(core_axis_name="c", subcore_axis_name="s")
    params = pltpu.CompilerParams(needs_layout_passes=False)
    dispatch = pl.kernel(
        _dispatch_body,
        out_type=jax.ShapeDtypeStruct((SLOTS_PAD, D), jnp.float32),
        mesh=mesh,
        compiler_params=params,
        scratch_types=[
            pltpu.VMEM((TOK_PER_TILE,), jnp.int32),
            pltpu.VMEM((TOK_PER_TILE, D), jnp.float32),
            pltpu.SemaphoreType.DMA,
            pltpu.SemaphoreType.DMA,
        ],
    )
    combine = pl.kernel(
        _combine_body,
        out_type=jax.ShapeDtypeStruct((T, D), jnp.float32),
        mesh=mesh,
        compiler_params=params,
        scratch_types=[
            pltpu.VMEM((TOK_PER_TILE,), jnp.int32),
            pltpu.VMEM((TOK_PER_TILE, D), jnp.float32),
            pltpu.SemaphoreType.DMA,
        ],
    )
    return dispatch, combine


def _dispatch(h, locs):
    return _sc_kernels()[0](h, locs)


def _combine(eo, locg):
    return _sc_kernels()[1](eo, locg)


def _final_body(h_ref, comb_ref, gate_ref, y_ref, o_ref, acc_ref):
    i = pl.program_id(0)
    g = gate_ref[...]
    hid = h_ref[...] + jnp.where(g > 0.0, comb_ref[...] * g, 0.0)
    cs = jnp.sum(hid, axis=0, keepdims=True)
    rows8 = lax.broadcasted_iota(jnp.int32, (NCHUNK, 1), 0)
    upd = jnp.where(rows8 == i, 1.0, 0.0) * cs

    @pl.when(i == 0)
    def _():
        acc_ref[...] = upd

    @pl.when(i > 0)
    def _():
        acc_ref[...] = acc_ref[...] + upd

    @pl.when(i == NCHUNK - 1)
    def _():
        sums = acc_ref[...]
        b0 = jnp.sum(sums[0:4, :], axis=0, keepdims=True)
        b1 = jnp.sum(sums[4:8, :], axis=0, keepdims=True)
        sent = jnp.concatenate([b0, b1], axis=0) * (1.0 / S)
        m = jnp.max(sent, axis=-1, keepdims=True)
        lse = jnp.log(jnp.sum(jnp.exp(sent - m), axis=-1, keepdims=True)) + m
        logp = sent - lse
        col = lax.broadcasted_iota(jnp.int32, (B, D), 1)
        row = lax.broadcasted_iota(jnp.int32, (B, D), 0)
        ysel = jnp.where(row == 0, y_ref[0], y_ref[1])
        loss = -jnp.sum(jnp.where(col == ysel, logp, 0.0)) * (1.0 / B)
        o_ref[...] = jnp.full((8, 128), loss, jnp.float32)


def _final(h, comb3, gate3, y32):
    return pl.pallas_call(
        _final_body,
        grid=(NCHUNK,),
        in_specs=[
            pl.BlockSpec((CHUNK, D), lambda i: (i, 0)),
            pl.BlockSpec((CHUNK, D), lambda i: (i, 0)),
            pl.BlockSpec((CHUNK, 1), lambda i: (i, 0)),
            pl.BlockSpec(memory_space=pltpu.SMEM),
        ],
        out_specs=pl.BlockSpec((8, 128), lambda i: (0, 0)),
        out_shape=jax.ShapeDtypeStruct((8, 128), jnp.float32),
        scratch_shapes=[pltpu.VMEM((NCHUNK, D), jnp.float32)],
    )(h, comb3, gate3, y32)


def kernel(x, y, W1, b1, Wg2, We2, be2, Wg3, We3, be3):
    x2d = x.reshape(T, D)
    y32 = y.astype(jnp.int32)

    h = _linear(x2d, W1, b1)

    locs2, locg2, gate2 = _route2(h, Wg2)
    buf2 = _dispatch(h, locs2.reshape(T))
    eo2 = _expert(buf2, We2, be2)
    comb2 = _combine(eo2.reshape(SLOTS, D), locg2.reshape(T))

    locs3, locg3, gate3, h2 = _route3(comb2, gate2, Wg3)
    buf3 = _dispatch(h2, locs3.reshape(T))
    eo3 = _expert(buf3, We3, be3)
    comb3 = _combine(eo3.reshape(SLOTS, D), locg3.reshape(T))

    loss = _final(h, comb3, gate3, y32)
    return loss[0, 0]


# speedup vs baseline: 1.9463x; 1.0324x over previous
import functools

import jax
import jax.numpy as jnp
from jax import lax
from jax.experimental import pallas as pl
from jax.experimental.pallas import tpu as pltpu
from jax.experimental.pallas import tpu_sc as plsc

B, S, D, E, CAP = 2, 2048, 768, 64, 80
T = B * S
SLOTS = E * CAP
TRASH = SLOTS
SLOTS_PAD = SLOTS + CAP
CHUNK = 512
NCHUNK = T // CHUNK

NW = 32
TOK_PER_TILE = T // NW


def _route_core(i, h, wg_ref, locs_ref, locg_ref, gate_ref, run_ref):
    logits = jnp.dot(h, wg_ref[...], preferred_element_type=jnp.float32)
    m = jnp.max(logits, axis=-1, keepdims=True)
    pmax = 1.0 / jnp.sum(jnp.exp(logits - m), axis=-1, keepdims=True)
    eidx = lax.broadcasted_iota(jnp.int32, (CHUNK, E), 1)
    idx = jnp.min(jnp.where(logits == m, eidx, E), axis=-1, keepdims=True)
    maskf = (eidx == idx).astype(jnp.float32)
    r = lax.broadcasted_iota(jnp.int32, (CHUNK, CHUNK), 0)
    c = lax.broadcasted_iota(jnp.int32, (CHUNK, CHUNK), 1)
    tril = (r >= c).astype(jnp.float32)

    @pl.when(i == 0)
    def _():
        run_ref[...] = jnp.zeros_like(run_ref)

    cum = (
        jnp.dot(tril, maskf, preferred_element_type=jnp.float32)
        + run_ref[...]
    )
    pos = jnp.sum(maskf * cum, axis=-1, keepdims=True) - 1.0
    run_ref[...] = run_ref[...] + jnp.sum(maskf, axis=0, keepdims=True)
    keep = pos < float(CAP)
    loc = idx * CAP + pos.astype(jnp.int32)
    tid = lax.broadcasted_iota(jnp.int32, (CHUNK, 1), 0)
    locs_ref[...] = jnp.where(keep, loc, TRASH + (tid & 15))
    locg_ref[...] = jnp.where(keep, loc, 0)
    gate_ref[...] = jnp.where(keep, pmax, 0.0)


def _route2_body(x_ref, w1_ref, b1_ref, wg_ref,
                 h_ref, locs_ref, locg_ref, gate_ref, run_ref):
    h = (
        jnp.dot(x_ref[...], w1_ref[...], preferred_element_type=jnp.float32)
        + b1_ref[...]
    )
    h_ref[...] = h
    _route_core(pl.program_id(0), h, wg_ref,
                locs_ref, locg_ref, gate_ref, run_ref)


def _route3_body(comb_ref, gprev_ref, wg_ref,
                 locs_ref, locg_ref, gate_ref, h2_ref, run_ref):
    g = gprev_ref[...]
    h = jnp.where(g > 0.0, comb_ref[...] * g, 0.0)
    h2_ref[...] = h
    _route_core(pl.program_id(0), h, wg_ref,
                locs_ref, locg_ref, gate_ref, run_ref)


_TOKVEC_SPEC = pl.BlockSpec((CHUNK, 1), lambda i: (i, 0))
_TOKVEC_I32 = jax.ShapeDtypeStruct((T, 1), jnp.int32)
_TOKVEC_F32 = jax.ShapeDtypeStruct((T, 1), jnp.float32)


def _route2(x2d, W1, b1, Wg):
    return pl.pallas_call(
        _route2_body,
        grid=(NCHUNK,),
        in_specs=[
            pl.BlockSpec((CHUNK, D), lambda i: (i, 0)),
            pl.BlockSpec((D, D), lambda i: (0, 0)),
            pl.BlockSpec((1, D), lambda i: (0, 0)),
            pl.BlockSpec((D, E), lambda i: (0, 0)),
        ],
        out_specs=[pl.BlockSpec((CHUNK, D), lambda i: (i, 0)),
                   _TOKVEC_SPEC, _TOKVEC_SPEC, _TOKVEC_SPEC],
        out_shape=[jax.ShapeDtypeStruct((T, D), jnp.float32),
                   _TOKVEC_I32, _TOKVEC_I32, _TOKVEC_F32],
        scratch_shapes=[pltpu.VMEM((1, E), jnp.float32)],
    )(x2d, W1, b1.reshape(1, D), Wg)


def _route3(comb, gate_prev, Wg):
    return pl.pallas_call(
        _route3_body,
        grid=(NCHUNK,),
        in_specs=[
            pl.BlockSpec((CHUNK, D), lambda i: (i, 0)),
            pl.BlockSpec((CHUNK, 1), lambda i: (i, 0)),
            pl.BlockSpec((D, E), lambda i: (0, 0)),
        ],
        out_specs=[_TOKVEC_SPEC, _TOKVEC_SPEC, _TOKVEC_SPEC,
                   pl.BlockSpec((CHUNK, D), lambda i: (i, 0))],
        out_shape=[_TOKVEC_I32, _TOKVEC_I32, _TOKVEC_F32,
                   jax.ShapeDtypeStruct((T, D), jnp.float32)],
        scratch_shapes=[pltpu.VMEM((1, E), jnp.float32)],
    )(comb, gate_prev, Wg)


def _expert_body(buf_ref, w_ref, b_ref, o_ref):
    o_ref[0] = (
        jnp.dot(buf_ref[0], w_ref[0], preferred_element_type=jnp.float32)
        + b_ref[0]
    )


def _expert(buf, We, be):
    buf = buf.reshape(SLOTS_PAD // CAP, CAP, D)
    return pl.pallas_call(
        _expert_body,
        grid=(E,),
        in_specs=[
            pl.BlockSpec((1, CAP, D), lambda e: (e, 0, 0)),
            pl.BlockSpec((1, D, D), lambda e: (e, 0, 0)),
            pl.BlockSpec((1, 1, D), lambda e: (e, 0, 0)),
        ],
        out_specs=pl.BlockSpec((1, CAP, D), lambda e: (e, 0, 0)),
        out_shape=jax.ShapeDtypeStruct((E, CAP, D), jnp.float32),
    )(buf, We, be.reshape(E, 1, D))


def _dispatch_body(h_hbm, locs_hbm, buf_hbm, loc_v, rows_v, sem, sem2):
    cid = lax.axis_index("c")
    sid = lax.axis_index("s")
    wid = sid * 2 + cid
    base = wid * TOK_PER_TILE
    pltpu.sync_copy(locs_hbm.at[pl.ds(base, TOK_PER_TILE)], loc_v)
    pltpu.async_copy(h_hbm.at[pl.ds(base, TOK_PER_TILE)], rows_v, sem).wait()
    pltpu.async_copy(rows_v, buf_hbm.at[loc_v], sem2).wait()


def _combine_body(eo_hbm, locg_hbm, out_hbm, idx_v, rows_v, sem):
    wid = lax.axis_index("s") * 2 + lax.axis_index("c")
    base = wid * TOK_PER_TILE
    pltpu.sync_copy(locg_hbm.at[pl.ds(base, TOK_PER_TILE)], idx_v)
    pltpu.async_copy(eo_hbm.at[idx_v], rows_v, sem).wait()
    pltpu.sync_copy(rows_v, out_hbm.at[pl.ds(base, TOK_PER_TILE)])


@functools.cache
def _sc_kernels():
    mesh = plsc.VectorSubcoreMesh(core_axis_name="c", subcore_axis_name="s")
    params = pltpu.CompilerParams(needs_layout_passes=False)
    dispatch = pl.kernel(
        _dispatch_body,
        out_type=jax.ShapeDtypeStruct((SLOTS_PAD, D), jnp.float32),
        mesh=mesh,
        compiler_params=params,
        scratch_types=[
            pltpu.VMEM((TOK_PER_TILE,), jnp.int32),
            pltpu.VMEM((TOK_PER_TILE, D), jnp.float32),
            pltpu.SemaphoreType.DMA,
            pltpu.SemaphoreType.DMA,
        ],
    )
    combine = pl.kernel(
        _combine_body,
        out_type=jax.ShapeDtypeStruct((T, D), jnp.float32),
        mesh=mesh,
        compiler_params=params,
        scratch_types=[
            pltpu.VMEM((TOK_PER_TILE,), jnp.int32),
            pltpu.VMEM((TOK_PER_TILE, D), jnp.float32),
            pltpu.SemaphoreType.DMA,
        ],
    )
    return dispatch, combine


def _dispatch(h, locs):
    return _sc_kernels()[0](h, locs)


def _combine(eo, locg):
    return _sc_kernels()[1](eo, locg)


def _final_body(h_ref, comb_ref, gate_ref, y_ref, o_ref, acc_ref):
    i = pl.program_id(0)
    g = gate_ref[...]
    hid = h_ref[...] + jnp.where(g > 0.0, comb_ref[...] * g, 0.0)
    cs = jnp.sum(hid, axis=0, keepdims=True)
    rows8 = lax.broadcasted_iota(jnp.int32, (NCHUNK, 1), 0)
    upd = jnp.where(rows8 == i, 1.0, 0.0) * cs

    @pl.when(i == 0)
    def _():
        acc_ref[...] = upd

    @pl.when(i > 0)
    def _():
        acc_ref[...] = acc_ref[...] + upd

    @pl.when(i == NCHUNK - 1)
    def _():
        sums = acc_ref[...]
        b0 = jnp.sum(sums[0:4, :], axis=0, keepdims=True)
        b1 = jnp.sum(sums[4:8, :], axis=0, keepdims=True)
        sent = jnp.concatenate([b0, b1], axis=0) * (1.0 / S)
        m = jnp.max(sent, axis=-1, keepdims=True)
        lse = jnp.log(jnp.sum(jnp.exp(sent - m), axis=-1, keepdims=True)) + m
        logp = sent - lse
        col = lax.broadcasted_iota(jnp.int32, (B, D), 1)
        row = lax.broadcasted_iota(jnp.int32, (B, D), 0)
        ysel = jnp.where(row == 0, y_ref[0], y_ref[1])
        loss = -jnp.sum(jnp.where(col == ysel, logp, 0.0)) * (1.0 / B)
        o_ref[...] = jnp.full((8, 128), loss, jnp.float32)


def _final(h, comb3, gate3, y32):
    return pl.pallas_call(
        _final_body,
        grid=(NCHUNK,),
        in_specs=[
            pl.BlockSpec((CHUNK, D), lambda i: (i, 0)),
            pl.BlockSpec((CHUNK, D), lambda i: (i, 0)),
            pl.BlockSpec((CHUNK, 1), lambda i: (i, 0)),
            pl.BlockSpec(memory_space=pltpu.SMEM),
        ],
        out_specs=pl.BlockSpec((8, 128), lambda i: (0, 0)),
        out_shape=jax.ShapeDtypeStruct((8, 128), jnp.float32),
        scratch_shapes=[pltpu.VMEM((NCHUNK, D), jnp.float32)],
    )(h, comb3, gate3, y32)


def kernel(x, y, W1, b1, Wg2, We2, be2, Wg3, We3, be3):
    x2d = x.reshape(T, D)
    y32 = y.astype(jnp.int32)

    h, locs2, locg2, gate2 = _route2(x2d, W1, b1, Wg2)
    buf2 = _dispatch(h, locs2.reshape(T))
    eo2 = _expert(buf2, We2, be2)
    comb2 = _combine(eo2.reshape(SLOTS, D), locg2.reshape(T))

    locs3, locg3, gate3, h2 = _route3(comb2, gate2, Wg3)
    buf3 = _dispatch(h2, locs3.reshape(T))
    eo3 = _expert(buf3, We3, be3)
    comb3 = _combine(eo3.reshape(SLOTS, D), locg3.reshape(T))

    loss = _final(h, comb3, gate3, y32)
    return loss[0, 0]
